# baseline (speedup 1.0000x reference)
import functools
import jax
import jax.numpy as jnp
from jax import lax
from jax.experimental import pallas as pl
from jax.experimental.pallas import tpu as pltpu
from jax.experimental.pallas import tpu_sc as plsc

B, N, M = 8, 2048, 2048
NC, NS, L = 2, 16, 16
NW = NC * NS
QPB = NW // B
RPW = N // QPB
UB = 4
CH = RPW // (UB * L)
JT = 16


def _sc_chamfer(a_hbm, b_hbm, out_hbm,
                rax, ray, raz, rbx, rby, rbz,
                c2a, mxa, mya, mza, c2b, mxb, myb, mzb, tot_v):
    wid = lax.axis_index("s") * NC + lax.axis_index("c")
    bi = wid // QPB
    q = wid % QPB

    ab = bi * 3 * N
    pltpu.sync_copy(a_hbm.at[pl.ds(ab, N)], rax)
    pltpu.sync_copy(a_hbm.at[pl.ds(ab + N, N)], ray)
    pltpu.sync_copy(a_hbm.at[pl.ds(ab + 2 * N, N)], raz)
    pltpu.sync_copy(b_hbm.at[pl.ds(ab, M)], rbx)
    pltpu.sync_copy(b_hbm.at[pl.ds(ab + M, M)], rby)
    pltpu.sync_copy(b_hbm.at[pl.ds(ab + 2 * M, M)], rbz)

    def build_quads(t, carry):
        s = pl.ds(t * L, L)
        for rx, ry, rz, c2, mx, my, mz in (
            (rax, ray, raz, c2a, mxa, mya, mza),
            (rbx, rby, rbz, c2b, mxb, myb, mzb),
        ):
            xv, yv, zv = rx[s], ry[s], rz[s]
            c2[s] = xv * xv + yv * yv + zv * zv
            mx[s] = -2.0 * xv
            my[s] = -2.0 * yv
            mz[s] = -2.0 * zv
        return carry

    lax.fori_loop(0, N // L, build_quads, 0)

    total = jnp.zeros((L,), jnp.float32)
    for (rvx, rvy, rvz), (c2s, mxs, mys, mzs) in (
        ((rax, ray, raz), (c2b, mxb, myb, mzb)),
        ((rbx, rby, rbz), (c2a, mxa, mya, mza)),
    ):
        def chunk_body(ch, tot):
            axs, ays, azs, a2s = [], [], [], []
            for u in range(UB):
                i0 = q * RPW + ch * UB * L + u * L
                s = pl.ds(i0, L)
                x, y, z = rvx[s], rvy[s], rvz[s]
                axs.append(x)
                ays.append(y)
                azs.append(z)
                a2s.append(x * x + y * y + z * z)

            def jbody(t, carry):
                s = pl.ds(t * JT, JT)
                c2v, mxv, myv, mzv = c2s[s], mxs[s], mys[s], mzs[s]
                ms = list(carry)
                for jj in range(JT):
                    c2, mx, my, mz = c2v[jj], mxv[jj], myv[jj], mzv[jj]
                    for u in range(UB):
                        e = c2 + axs[u] * mx + ays[u] * my + azs[u] * mz
                        ms[u] = jnp.minimum(ms[u], e)
                return tuple(ms)

            init = tuple(jnp.full((L,), jnp.inf, jnp.float32) for _ in range(UB))
            ms = lax.fori_loop(0, M // JT, jbody, init)
            for u in range(UB):
                tot = tot + jnp.maximum(ms[u] + a2s[u], 0.0)
            return tot

        total = lax.fori_loop(0, CH, chunk_body, total)

    tot_v[...] = total
    pltpu.sync_copy(tot_v, out_hbm.at[pl.ds(wid * L, L)])


def _sc_call(a_t, b_t):
    return pl.kernel(
        _sc_chamfer,
        out_type=jax.ShapeDtypeStruct((NW * L,), jnp.float32),
        mesh=plsc.VectorSubcoreMesh(core_axis_name="c", subcore_axis_name="s"),
        scratch_types=(
            [pltpu.VMEM((N,), jnp.float32) for _ in range(6)]
            + [pltpu.VMEM((N,), jnp.float32) for _ in range(8)]
            + [pltpu.VMEM((L,), jnp.float32)]
        ),
    )(a_t, b_t)


def kernel(array1, array2):
    a_t = jnp.transpose(array1, (0, 2, 1)).reshape(B * 3 * N)
    b_t = jnp.transpose(array2, (0, 2, 1)).reshape(B * 3 * M)
    out = _sc_call(a_t, b_t)
    return jnp.sum(out) * (1.0 / (B * N))


# speedup vs baseline: 1.2624x; 1.2624x over previous
import functools
import jax
import jax.numpy as jnp
from jax import lax
from jax.experimental import pallas as pl
from jax.experimental.pallas import tpu as pltpu
from jax.experimental.pallas import tpu_sc as plsc

B, N, M = 8, 2048, 2048
NC, NS, L = 2, 16, 16
NW = NC * NS
QPB = NW // B
RPW = N // QPB
UB = 4
CH = RPW // (UB * L)
JT = 16

_BCAST_DNUMS = lax.GatherDimensionNumbers(
    offset_dims=(), collapsed_slice_dims=(0,), start_index_map=(0,))


def _bcast(vec, lane_idx):
    return lax.gather(vec, lane_idx[:, None], _BCAST_DNUMS, slice_sizes=(1,),
                      mode=lax.GatherScatterMode.PROMISE_IN_BOUNDS)


def _sc_chamfer(a_hbm, b_hbm, out_hbm,
                rax, ray, raz, rbx, rby, rbz,
                c2a, mxa, mya, mza, c2b, mxb, myb, mzb, tot_v):
    wid = lax.axis_index("s") * NC + lax.axis_index("c")
    bi = wid // QPB
    q = wid % QPB

    ab = bi * 3 * N
    pltpu.sync_copy(a_hbm.at[pl.ds(ab, N)], rax)
    pltpu.sync_copy(a_hbm.at[pl.ds(ab + N, N)], ray)
    pltpu.sync_copy(a_hbm.at[pl.ds(ab + 2 * N, N)], raz)
    pltpu.sync_copy(b_hbm.at[pl.ds(ab, M)], rbx)
    pltpu.sync_copy(b_hbm.at[pl.ds(ab + M, M)], rby)
    pltpu.sync_copy(b_hbm.at[pl.ds(ab + 2 * M, M)], rbz)

    def build_quads(t, carry):
        s = pl.ds(t * L, L)
        for rx, ry, rz, c2, mx, my, mz in (
            (rax, ray, raz, c2a, mxa, mya, mza),
            (rbx, rby, rbz, c2b, mxb, myb, mzb),
        ):
            xv, yv, zv = rx[s], ry[s], rz[s]
            c2[s] = xv * xv + yv * yv + zv * zv
            mx[s] = -2.0 * xv
            my[s] = -2.0 * yv
            mz[s] = -2.0 * zv
        return carry

    lax.fori_loop(0, N // L, build_quads, 0)

    total = jnp.zeros((L,), jnp.float32)
    for (rvx, rvy, rvz), (c2s, mxs, mys, mzs) in (
        ((rax, ray, raz), (c2b, mxb, myb, mzb)),
        ((rbx, rby, rbz), (c2a, mxa, mya, mza)),
    ):
        def chunk_body(ch, tot):
            axs, ays, azs, a2s = [], [], [], []
            for u in range(UB):
                i0 = q * RPW + ch * UB * L + u * L
                s = pl.ds(i0, L)
                x, y, z = rvx[s], rvy[s], rvz[s]
                axs.append(x)
                ays.append(y)
                azs.append(z)
                a2s.append(x * x + y * y + z * z)

            def jbody(t, carry):
                s = pl.ds(t * JT, JT)
                c2v, mxv, myv, mzv = c2s[s], mxs[s], mys[s], mzs[s]
                ms = list(carry)
                for jj in range(JT):
                    lane = jnp.full((L,), jj, jnp.int32)
                    c2 = _bcast(c2v, lane)
                    mx = _bcast(mxv, lane)
                    my = _bcast(myv, lane)
                    mz = _bcast(mzv, lane)
                    for u in range(UB):
                        e = c2 + axs[u] * mx + ays[u] * my + azs[u] * mz
                        ms[u] = jnp.minimum(ms[u], e)
                return tuple(ms)

            init = tuple(jnp.full((L,), jnp.inf, jnp.float32) for _ in range(UB))
            ms = lax.fori_loop(0, M // JT, jbody, init)
            for u in range(UB):
                tot = tot + jnp.maximum(ms[u] + a2s[u], 0.0)
            return tot

        total = lax.fori_loop(0, CH, chunk_body, total)

    tot_v[...] = total
    pltpu.sync_copy(tot_v, out_hbm.at[pl.ds(wid * L, L)])


def _sc_call(a_t, b_t):
    return pl.kernel(
        _sc_chamfer,
        out_type=jax.ShapeDtypeStruct((NW * L,), jnp.float32),
        mesh=plsc.VectorSubcoreMesh(core_axis_name="c", subcore_axis_name="s"),
        scratch_types=(
            [pltpu.VMEM((N,), jnp.float32) for _ in range(6)]
            + [pltpu.VMEM((N,), jnp.float32) for _ in range(8)]
            + [pltpu.VMEM((L,), jnp.float32)]
        ),
    )(a_t, b_t)


def kernel(array1, array2):
    a_t = jnp.transpose(array1, (0, 2, 1)).reshape(B * 3 * N)
    b_t = jnp.transpose(array2, (0, 2, 1)).reshape(B * 3 * M)
    out = _sc_call(a_t, b_t)
    return jnp.sum(out) * (1.0 / (B * N))


# speedup vs baseline: 2.4033x; 1.9037x over previous
import jax
import jax.numpy as jnp
from jax import lax
from jax.experimental import pallas as pl
from jax.experimental.pallas import tpu as pltpu

B, N, M = 8, 2048, 2048
K = 5
RC = 8


def _chamfer_body(l_ref, r_ref, out_ref, d_ref):
    bi = pl.program_id(0)
    l = l_ref[0]
    r = r_ref[0]
    d_ref[...] = lax.dot_general(l, r, (((0,), (0,)), ((), ())),
                                 preferred_element_type=jnp.float32)

    def body(i, carry):
        colacc, s1 = carry
        row = d_ref[pl.ds(i * RC, RC), :]
        rmin = jnp.min(row, axis=1)
        s1 = s1 + jnp.sum(jnp.maximum(rmin, 0.0))
        colacc = jnp.minimum(colacc, row)
        return colacc, s1

    colacc0 = jnp.full((RC, M), jnp.inf, jnp.float32)
    colacc, s1 = lax.fori_loop(0, N // RC, body, (colacc0, jnp.float32(0.0)))
    s2 = jnp.sum(jnp.maximum(jnp.min(colacc, axis=0), 0.0))
    inc = jnp.reshape(s1 + s2, (1, 1))

    @pl.when(bi == 0)
    def _init():
        out_ref[...] = inc

    @pl.when(bi > 0)
    def _acc():
        out_ref[...] += inc


def kernel(array1, array2):
    a_t = jnp.transpose(array1, (0, 2, 1))
    b_t = jnp.transpose(array2, (0, 2, 1))
    a2 = jnp.sum(a_t * a_t, axis=1, keepdims=True)
    b2 = jnp.sum(b_t * b_t, axis=1, keepdims=True)
    ones_a = jnp.ones_like(a2)
    l_aug = jnp.concatenate([a_t, a2, ones_a], axis=1)
    r_aug = jnp.concatenate([-2.0 * b_t, ones_a, b2], axis=1)
    out = pl.pallas_call(
        _chamfer_body,
        grid=(B,),
        in_specs=[
            pl.BlockSpec((1, K, N), lambda i: (i, 0, 0)),
            pl.BlockSpec((1, K, M), lambda i: (i, 0, 0)),
        ],
        out_specs=pl.BlockSpec((1, 1), lambda i: (0, 0)),
        out_shape=jax.ShapeDtypeStruct((1, 1), jnp.float32),
        scratch_shapes=[pltpu.VMEM((N, M), jnp.float32)],
    )(l_aug, r_aug)
    return out[0, 0] * (1.0 / (B * N))


# speedup vs baseline: 36.5471x; 15.2071x over previous
import jax
import jax.numpy as jnp
from jax import lax
from jax.experimental import pallas as pl
from jax.experimental.pallas import tpu as pltpu

B, N, M = 8, 2048, 2048
K = 5
CB = 4
MB = M // CB


def _chamfer_body(l_ref, r_ref, out_ref):
    bi = pl.program_id(0)
    l = l_ref[0]
    r = r_ref[0]

    s2 = jnp.float32(0.0)
    rowacc = None
    for cb in range(CB):
        rblk = r[:, cb * MB:(cb + 1) * MB]
        dblk = lax.dot_general(l, rblk, (((0,), (0,)), ((), ())),
                               preferred_element_type=jnp.float32)
        rm = jnp.min(dblk, axis=1)
        rowacc = rm if rowacc is None else jnp.minimum(rowacc, rm)
        s2 = s2 + jnp.sum(jnp.maximum(jnp.min(dblk, axis=0), 0.0))
    s1 = jnp.sum(jnp.maximum(rowacc, 0.0))
    inc = jnp.reshape(s1 + s2, (1, 1))

    @pl.when(bi == 0)
    def _init():
        out_ref[...] = inc

    @pl.when(bi > 0)
    def _acc():
        out_ref[...] += inc


def kernel(array1, array2):
    a_t = jnp.transpose(array1, (0, 2, 1))
    b_t = jnp.transpose(array2, (0, 2, 1))
    a2 = jnp.sum(a_t * a_t, axis=1, keepdims=True)
    b2 = jnp.sum(b_t * b_t, axis=1, keepdims=True)
    ones_a = jnp.ones_like(a2)
    l_aug = jnp.concatenate([a_t, a2, ones_a], axis=1)
    r_aug = jnp.concatenate([-2.0 * b_t, ones_a, b2], axis=1)
    out = pl.pallas_call(
        _chamfer_body,
        grid=(B,),
        in_specs=[
            pl.BlockSpec((1, K, N), lambda i: (i, 0, 0)),
            pl.BlockSpec((1, K, M), lambda i: (i, 0, 0)),
        ],
        out_specs=pl.BlockSpec((1, 1), lambda i: (0, 0)),
        out_shape=jax.ShapeDtypeStruct((1, 1), jnp.float32),
    )(l_aug, r_aug)
    return out[0, 0] * (1.0 / (B * N))


# speedup vs baseline: 39.6503x; 1.0849x over previous
import jax
import jax.numpy as jnp
from jax import lax
from jax.experimental import pallas as pl
from jax.experimental.pallas import tpu as pltpu

B, N, M = 8, 2048, 2048
K = 5
CB = 4
MB = M // CB


def _chamfer_body(l_ref, r_ref, out_ref):
    acc = jnp.float32(0.0)
    for bi in range(B):
        l = l_ref[bi]
        r = r_ref[bi]
        s2 = jnp.float32(0.0)
        rowacc = None
        for cb in range(CB):
            rblk = r[:, cb * MB:(cb + 1) * MB]
            dblk = lax.dot_general(l, rblk, (((0,), (0,)), ((), ())),
                                   preferred_element_type=jnp.float32)
            rm = jnp.min(dblk, axis=1)
            rowacc = rm if rowacc is None else jnp.minimum(rowacc, rm)
            s2 = s2 + jnp.sum(jnp.maximum(jnp.min(dblk, axis=0), 0.0))
        s1 = jnp.sum(jnp.maximum(rowacc, 0.0))
        acc = acc + s1 + s2
    out_ref[...] = jnp.reshape(acc, (1, 1))


def kernel(array1, array2):
    a_t = jnp.transpose(array1, (0, 2, 1))
    b_t = jnp.transpose(array2, (0, 2, 1))
    a2 = jnp.sum(a_t * a_t, axis=1, keepdims=True)
    b2 = jnp.sum(b_t * b_t, axis=1, keepdims=True)
    ones_a = jnp.ones_like(a2)
    l_aug = jnp.concatenate([a_t, a2, ones_a], axis=1)
    r_aug = jnp.concatenate([-2.0 * b_t, ones_a, b2], axis=1)
    out = pl.pallas_call(
        _chamfer_body,
        out_shape=jax.ShapeDtypeStruct((1, 1), jnp.float32),
    )(l_aug, r_aug)
    return out[0, 0] * (1.0 / (B * N))
